# Initial kernel scaffold; baseline (speedup 1.0000x reference)
import jax
import jax.numpy as jnp
from jax.experimental import pallas as pl


def kernel(hidden_states, wq, wk, wv, wo):
    raise NotImplementedError("write your pallas kernel here")


# speedup vs baseline: 3.4225x; 3.4225x over previous
import functools
import math

import jax
import jax.numpy as jnp
from jax.experimental import pallas as pl
from jax.experimental.pallas import tpu as pltpu

_VMEM_LIMIT = 64 * 1024 * 1024


def _fused_mha_kernel(x_ref, wq_ref, wk_ref, wv_ref, wo_ref, o_ref,
                      *, num_heads, head_dim):
    f32 = jnp.float32
    x = x_ref[...]
    cdt = x.dtype

    q = jnp.dot(x, wq_ref[...], preferred_element_type=f32).astype(cdt)
    k = jnp.dot(x, wk_ref[...], preferred_element_type=f32).astype(cdt)
    v = jnp.dot(x, wv_ref[...], preferred_element_type=f32).astype(cdt)

    outs = []
    for h in range(num_heads):
        sl = slice(h * head_dim, (h + 1) * head_dim)
        qh, kh, vh = q[:, sl], k[:, sl], v[:, sl]
        s = jax.lax.dot_general(
            qh, kh, (((1,), (1,)), ((), ())),
            preferred_element_type=f32)
        m = jnp.max(s, axis=-1, keepdims=True)
        p = jnp.exp(s - m)
        l = jnp.sum(p, axis=-1, keepdims=True)
        acc = jnp.dot(p.astype(cdt), vh, preferred_element_type=f32)
        outs.append((acc * pl.reciprocal(l, approx=False)).astype(cdt))

    attn = jnp.concatenate(outs, axis=-1)

    out = jnp.dot(attn, wo_ref[...], preferred_element_type=f32)
    o_ref[...] = out.astype(cdt).astype(o_ref.dtype)


def kernel(hidden_states, wq, wk, wv, wo):
    B, T, E = hidden_states.shape
    num_heads = 16
    head_dim = E // num_heads
    scaling = head_dim ** (-0.5)
    orig_dtype = hidden_states.dtype
    cdt = jnp.bfloat16

    wq_t = (wq.astype(jnp.float32).T * scaling).astype(cdt)
    wk_t = wk.T.astype(cdt)
    wv_t = wv.T.astype(cdt)
    wo_t = wo.T.astype(cdt)
    x = hidden_states.astype(cdt)

    itemsize = 2
    cost = pl.CostEstimate(
        flops=2 * B * T * E * E * 4 + 4 * B * num_heads * T * T * head_dim,
        transcendentals=B * num_heads * T * T,
        bytes_accessed=(B * T * E * 2 + 4 * E * E) * itemsize
                        + B * T * E * 4,
    )

    fused = functools.partial(
        _fused_mha_kernel, num_heads=num_heads, head_dim=head_dim)

    out = pl.pallas_call(
        fused,
        out_shape=jax.ShapeDtypeStruct((B, T, E), orig_dtype),
        grid_spec=pltpu.PrefetchScalarGridSpec(
            num_scalar_prefetch=0,
            grid=(B,),
            in_specs=[
                pl.BlockSpec((None, T, E), lambda b: (b, 0, 0)),
                pl.BlockSpec((E, E), lambda b: (0, 0)),
                pl.BlockSpec((E, E), lambda b: (0, 0)),
                pl.BlockSpec((E, E), lambda b: (0, 0)),
                pl.BlockSpec((E, E), lambda b: (0, 0)),
            ],
            out_specs=pl.BlockSpec((None, T, E), lambda b: (b, 0, 0)),
        ),
        compiler_params=pltpu.CompilerParams(
            dimension_semantics=("parallel",),
            vmem_limit_bytes=_VMEM_LIMIT,
        ),
        cost_estimate=cost,
    )(x, wq_t, wk_t, wv_t, wo_t)
    return out


# speedup vs baseline: 3.8660x; 1.1296x over previous
import functools

import jax
import jax.numpy as jnp
from jax.experimental import pallas as pl
from jax.experimental.pallas import tpu as pltpu

_VMEM_LIMIT = 64 * 1024 * 1024

_DN_T = (((1,), (1,)), ((), ()))


def _fused_mha_kernel(x_ref, wq_ref, wk_ref, wv_ref, wo_ref, o_ref,
                      *, num_heads, head_dim, scaling):
    f32 = jnp.float32
    cdt = jnp.bfloat16
    x = x_ref[...].astype(cdt)

    q = (jax.lax.dot_general(x, wq_ref[...], _DN_T,
                             preferred_element_type=f32)
         * scaling).astype(cdt)
    k = jax.lax.dot_general(x, wk_ref[...], _DN_T,
                            preferred_element_type=f32).astype(cdt)
    v = jax.lax.dot_general(x, wv_ref[...], _DN_T,
                            preferred_element_type=f32).astype(cdt)

    outs = []
    for h in range(num_heads):
        sl = slice(h * head_dim, (h + 1) * head_dim)
        qh, kh, vh = q[:, sl], k[:, sl], v[:, sl]
        s = jax.lax.dot_general(qh, kh, _DN_T,
                                preferred_element_type=f32)
        m = jnp.max(s, axis=-1, keepdims=True)
        p = jnp.exp(s - m)
        l = jnp.sum(p, axis=-1, keepdims=True)
        acc = jnp.dot(p.astype(cdt), vh, preferred_element_type=f32)
        outs.append((acc * pl.reciprocal(l, approx=False)).astype(cdt))

    attn = jnp.concatenate(outs, axis=-1)

    out = jax.lax.dot_general(attn, wo_ref[...], _DN_T,
                              preferred_element_type=f32)
    o_ref[...] = out.astype(cdt).astype(o_ref.dtype)


def kernel(hidden_states, wq, wk, wv, wo):
    B, T, E = hidden_states.shape
    num_heads = 16
    head_dim = E // num_heads
    scaling = head_dim ** (-0.5)
    orig_dtype = hidden_states.dtype
    cdt = jnp.bfloat16

    wq_c = wq.astype(cdt)
    wk_c = wk.astype(cdt)
    wv_c = wv.astype(cdt)
    wo_c = wo.astype(cdt)

    cost = pl.CostEstimate(
        flops=2 * B * T * E * E * 4 + 4 * B * num_heads * T * T * head_dim,
        transcendentals=B * num_heads * T * T,
        bytes_accessed=B * T * E * 8 + 4 * E * E * 2,
    )

    fused = functools.partial(
        _fused_mha_kernel, num_heads=num_heads, head_dim=head_dim,
        scaling=scaling)

    out = pl.pallas_call(
        fused,
        out_shape=jax.ShapeDtypeStruct((B, T, E), orig_dtype),
        grid_spec=pltpu.PrefetchScalarGridSpec(
            num_scalar_prefetch=0,
            grid=(B,),
            in_specs=[
                pl.BlockSpec((None, T, E), lambda b: (b, 0, 0)),
                pl.BlockSpec((E, E), lambda b: (0, 0)),
                pl.BlockSpec((E, E), lambda b: (0, 0)),
                pl.BlockSpec((E, E), lambda b: (0, 0)),
                pl.BlockSpec((E, E), lambda b: (0, 0)),
            ],
            out_specs=pl.BlockSpec((None, T, E), lambda b: (b, 0, 0)),
        ),
        compiler_params=pltpu.CompilerParams(
            dimension_semantics=("parallel",),
            vmem_limit_bytes=_VMEM_LIMIT,
        ),
        cost_estimate=cost,
    )(hidden_states, wq_c, wk_c, wv_c, wo_c)
    return out
